# baseline (speedup 1.0000x reference)
import functools

import jax
import jax.numpy as jnp
from jax import lax
from jax.experimental import pallas as pl
from jax.experimental.pallas import tpu as pltpu
from jax.experimental.pallas import tpu_sc as plsc

_V = 1_000_000
_D = 64
_B = 16384
_NC = 2
_NS = 16
_NW = _NC * _NS
_BPW = _B // _NW


def _gather_body(table_hbm, idx_hbm, out_hbm, idx_v, rows_v, sem):
    wid = lax.axis_index("s") * _NC + lax.axis_index("c")
    base = wid * _BPW
    pltpu.sync_copy(idx_hbm.at[pl.ds(base, _BPW)], idx_v)
    pltpu.async_copy(table_hbm.at[idx_v], rows_v, sem).wait()
    pltpu.sync_copy(rows_v, out_hbm.at[pl.ds(base, _BPW)])


def kernel(input, index):
    index = index.astype(jnp.int32)
    mesh = plsc.VectorSubcoreMesh(core_axis_name="c", subcore_axis_name="s")
    run = functools.partial(
        pl.kernel,
        mesh=mesh,
        out_type=jax.ShapeDtypeStruct((_B, _D), jnp.float32),
        scratch_types=[
            pltpu.VMEM((_BPW,), jnp.int32),
            pltpu.VMEM((_BPW, _D), jnp.float32),
            pltpu.SemaphoreType.DMA,
        ],
        compiler_params=pltpu.CompilerParams(use_tc_tiling_on_sc=False),
    )(_gather_body)
    return run(input, index)


# speedup vs baseline: 2.1018x; 2.1018x over previous
import functools

import jax
import jax.numpy as jnp
from jax import lax
from jax.experimental import pallas as pl
from jax.experimental.pallas import tpu as pltpu
from jax.experimental.pallas import tpu_sc as plsc

_V = 1_000_000
_D = 64
_B = 16384
_NW = 32
_WIN = 256
_NFULL = _V // _WIN
_TAILW = _NFULL
_CAP = _B + 16
_SENT = 0x3FFFFFFF


def _make_flush(chunk_ref, staging, ring_l, ring_i, out_hbm, osem, iota):
    def flush(_, carry):
        wr, rd, bctr = carry
        slot = lax.rem(bctr, 8)
        sbase = slot * 1024

        nreal = jnp.minimum(wr - rd, 16)
        ridx = lax.rem(rd + iota, 64)
        lv = plsc.load_gather(ring_l, [ridx])
        iv = plsc.load_gather(ring_i, [ridx])
        sel = iota < nreal
        lv = jnp.where(sel, lv, jnp.broadcast_to(lv[0], (16,)))
        iv = jnp.where(sel, iv, jnp.broadcast_to(iv[0], (16,)))
        dest = sbase + iota * 64
        for c in range(_D):
            vals = plsc.load_gather(
                chunk_ref,
                [
                    jnp.full((16,), c // 8, jnp.int32),
                    jnp.full((16,), c % 8, jnp.int32),
                    lv,
                ],
            )
            plsc.store_scatter(staging, [dest + c], vals)
        descs = []
        for j in range(16):
            off = pl.multiple_of(iv[j] * _D, _D)
            descs.append(pltpu.async_copy(
                staging.at[pl.ds(sbase + j * _D, _D)],
                out_hbm.at[pl.ds(off, _D)],
                osem.at[slot],
            ))
        for dsc in descs:
            dsc.wait()
        return wr, rd + nreal, bctr + 1

    return flush


def _scan_window(win, nchunks, carry, bucket_r, bucket_i, ring_l, ring_i,
                 flush, iota):
    def chunk_body(q, carry):
        wr, rd, bctr = carry
        qo = pl.multiple_of(q * 16, 16)
        rv = bucket_r[pl.ds(qo, 16)]
        wv = lax.shift_right_logical(rv, 8)
        mk = wv == win
        pc = jnp.sum(mk.astype(jnp.int32))

        @pl.when(pc > 0)
        def _store():
            pos = lax.rem(wr + jnp.cumsum(mk.astype(jnp.int32)) - 1, 64)
            iv = bucket_i[pl.ds(qo, 16)]
            plsc.store_scatter(ring_l, [pos], rv & 255, mask=mk)
            plsc.store_scatter(ring_i, [pos], iv, mask=mk)

        wr = wr + pc
        nfl = lax.shift_right_logical(wr - rd, 4)
        return lax.fori_loop(0, nfl, flush, (wr, rd, bctr))

    carry = lax.fori_loop(0, nchunks, chunk_body, carry)
    wr, rd, bctr = carry
    nfl = lax.shift_right_logical(wr - rd + 15, 4)
    return lax.fori_loop(0, nfl, flush, carry)


def _gather_body(table_hbm, idx_hbm, out_hbm, diag_hbm, idx_v, bucket_r,
                 bucket_i, chunks, tailc, ring_l, ring_i, staging, csem,
                 osem):
    w = lax.axis_index("s") * 2 + lax.axis_index("c")
    iota = lax.iota(jnp.int32, 16)
    pltpu.sync_copy(idx_hbm.at[pl.ds(0, _B)], idx_v)

    def p1(q, cnt):
        qo = pl.multiple_of(q * 16, 16)
        rv = idx_v[pl.ds(qo, 16)]
        mk = (lax.shift_right_logical(rv, 8) & (_NW - 1)) == w
        pc = jnp.sum(mk.astype(jnp.int32))

        @pl.when(pc > 0)
        def _store():
            pos = cnt + jnp.cumsum(mk.astype(jnp.int32)) - 1
            plsc.store_scatter(bucket_r, [pos], rv, mask=mk)
            plsc.store_scatter(bucket_i, [pos], q * 16 + iota, mask=mk)

        return cnt + pc

    cnt = lax.fori_loop(0, _B // 16, p1, 0)
    plsc.store_scatter(bucket_r, [cnt + iota],
                       jnp.full((16,), _SENT, jnp.int32))
    nchunks = lax.shift_right_logical(cnt + 15, 4)

    nt = lax.select(w < _NFULL - (_NFULL // _NW) * _NW,
                    _NFULL // _NW + 1, _NFULL // _NW)

    def chunk_slice(t):
        win = w + _NW * t
        r0 = pl.multiple_of(win * _WIN, 128)
        return table_hbm.at[:, :, pl.ds(r0, _WIN)]

    def fire(t):
        slot = lax.rem(t, 2)
        pltpu.async_copy(chunk_slice(t), chunks.at[slot], csem.at[slot])

    fire(0)

    def wbody(t, carry):
        win = w + _NW * t

        @pl.when(t + 1 < nt)
        def _prefetch():
            fire(t + 1)

        pltpu.make_async_copy(
            chunk_slice(t), chunks.at[lax.rem(t, 2)], csem.at[lax.rem(t, 2)]
        ).wait()
        flush = _make_flush(chunks.at[lax.rem(t, 2)], staging, ring_l,
                            ring_i, out_hbm, osem, iota)
        return _scan_window(win, nchunks, carry, bucket_r, bucket_i,
                            ring_l, ring_i, flush, iota)

    carry = lax.fori_loop(0, nt, wbody, (0, 0, 0))

    pltpu.sync_copy(
        table_hbm.at[:, :, pl.ds(_TAILW * _WIN, _V - _TAILW * _WIN)], tailc
    )
    flush_t = _make_flush(tailc, staging, ring_l, ring_i, out_hbm, osem,
                          iota)
    carry = _scan_window(_TAILW, nchunks, carry, bucket_r, bucket_i,
                         ring_l, ring_i, flush_t, iota)

    wr, rd, bctr = carry
    dv = jnp.zeros((16,), jnp.int32)
    for k, val in enumerate([cnt, wr, rd, bctr, nt, nchunks]):
        dv = jnp.where(iota == k, val, dv)
    ring_l[pl.ds(0, 16)] = dv
    pltpu.sync_copy(ring_l.at[pl.ds(0, 16)],
                    diag_hbm.at[pl.ds(pl.multiple_of(w * 16, 16), 16)])
    pltpu.sync_copy(diag_hbm.at[pl.ds(pl.multiple_of(w * 16, 16), 16)],
                    ring_i.at[pl.ds(0, 16)])


def kernel(input, index):
    table3 = jnp.reshape(jnp.swapaxes(input, 0, 1), (8, 8, _V))
    index = index.astype(jnp.int32)
    mesh = plsc.VectorSubcoreMesh(core_axis_name="c", subcore_axis_name="s")
    run = functools.partial(
        pl.kernel,
        mesh=mesh,
        out_type=(jax.ShapeDtypeStruct((_B * _D,), jnp.float32),
                  jax.ShapeDtypeStruct((_NW * 16,), jnp.int32)),
        scratch_types=[
            pltpu.VMEM((_B,), jnp.int32),
            pltpu.VMEM((_CAP,), jnp.int32),
            pltpu.VMEM((_CAP,), jnp.int32),
            pltpu.VMEM((2, 8, 8, _WIN), jnp.float32),
            pltpu.VMEM((8, 8, _V - _TAILW * _WIN), jnp.float32),
            pltpu.VMEM((64,), jnp.int32),
            pltpu.VMEM((64,), jnp.int32),
            pltpu.VMEM((8 * 16 * _D,), jnp.float32),
            pltpu.SemaphoreType.DMA((2,)),
            pltpu.SemaphoreType.DMA((8,)),
        ],
        compiler_params=pltpu.CompilerParams(needs_layout_passes=False),
    )(_gather_body)
    out1, _ = run(table3, index)
    return jnp.reshape(out1, (_B, _D))


# speedup vs baseline: 2.6967x; 1.2831x over previous
import functools

import jax
import jax.numpy as jnp
from jax import lax
from jax.experimental import pallas as pl
from jax.experimental.pallas import tpu as pltpu
from jax.experimental.pallas import tpu_sc as plsc

_V = 1_000_000
_D = 64
_B = 16384
_NW = 32
_WIN = 256
_NFULL = _V // _WIN
_TAILW = _NFULL
_CAP = _B + 16
_SENT = 0x3FFFFFFF


def _make_flush(chunk_ref, staging, ring_l, ring_i, out_hbm, osem, iota):
    def flush(_, carry):
        wr, rd, bctr = carry
        slot = lax.rem(bctr, 6)
        sbase = slot * 1024

        nreal = jnp.minimum(wr - rd, 16)
        ridx = lax.rem(rd + iota, 64)
        lv = plsc.load_gather(ring_l, [ridx])
        iv = plsc.load_gather(ring_i, [ridx])
        sel = iota < nreal
        lv = jnp.where(sel, lv, jnp.broadcast_to(lv[0], (16,)))
        iv = jnp.where(sel, iv, jnp.broadcast_to(iv[0], (16,)))
        dest = sbase + iota * 64
        for c in range(_D):
            vals = plsc.load_gather(
                chunk_ref,
                [
                    jnp.full((16,), c // 8, jnp.int32),
                    jnp.full((16,), c % 8, jnp.int32),
                    lv,
                ],
            )
            plsc.store_scatter(staging, [dest + c], vals)
        descs = []
        for j in range(16):
            off = pl.multiple_of(iv[j] * _D, _D)
            descs.append(pltpu.async_copy(
                staging.at[pl.ds(sbase + j * _D, _D)],
                out_hbm.at[pl.ds(off, _D)],
                osem.at[slot],
            ))
        for dsc in descs:
            dsc.wait()
        return wr, rd + nreal, bctr + 1

    return flush


def _scan_window(win, goff, gchunks, carry, bucket_r, bucket_i, ring_l,
                 ring_i, flush, iota):
    def chunk_body(q, carry):
        wr, rd, bctr = carry
        qo = pl.multiple_of(goff + q * 16, 16)
        rv = bucket_r[pl.ds(qo, 16)]
        wv = lax.shift_right_logical(rv, 8)
        mk = wv == win
        pc = jnp.sum(mk.astype(jnp.int32))

        @pl.when(pc > 0)
        def _store():
            pos = lax.rem(wr + jnp.cumsum(mk.astype(jnp.int32)) - 1, 64)
            iv = bucket_i[pl.ds(qo, 16)]
            plsc.store_scatter(ring_l, [pos], rv & 255, mask=mk)
            plsc.store_scatter(ring_i, [pos], iv, mask=mk)

        wr = wr + pc
        nfl = lax.shift_right_logical(wr - rd, 4)
        return lax.fori_loop(0, nfl, flush, (wr, rd, bctr))

    carry = lax.fori_loop(0, gchunks, chunk_body, carry)
    wr, rd, bctr = carry
    nfl = lax.shift_right_logical(wr - rd + 15, 4)
    return lax.fori_loop(0, nfl, flush, carry)


def _gather_body(table_hbm, idx_hbm, out_hbm, diag_hbm, idx_v, bucket_r,
                 bucket_i, bucket2_r, bucket2_i, chunks, tailc, ring_l,
                 ring_i, staging, csem, osem):
    w = lax.axis_index("s") * 2 + lax.axis_index("c")
    iota = lax.iota(jnp.int32, 16)
    pltpu.sync_copy(idx_hbm.at[pl.ds(0, _B)], idx_v)

    def p1(q, cnt):
        qo = pl.multiple_of(q * 16, 16)
        rv = idx_v[pl.ds(qo, 16)]
        mk = (lax.shift_right_logical(rv, 8) & (_NW - 1)) == w
        pc = jnp.sum(mk.astype(jnp.int32))

        @pl.when(pc > 0)
        def _store():
            pos = cnt + jnp.cumsum(mk.astype(jnp.int32)) - 1
            plsc.store_scatter(bucket_r, [pos], rv, mask=mk)
            plsc.store_scatter(bucket_i, [pos], q * 16 + iota, mask=mk)

        return cnt + pc

    cnt = lax.fori_loop(0, _B // 16, p1, 0)
    plsc.store_scatter(bucket_r, [cnt + iota],
                       jnp.full((16,), _SENT, jnp.int32))
    nchunks = lax.shift_right_logical(cnt + 15, 4)

    goffs, gchunks = [], []
    off = 0
    for g in range(8):
        def p1b(q, st, g=g):
            o, gc = st
            qo = pl.multiple_of(q * 16, 16)
            rv = bucket_r[pl.ds(qo, 16)]
            mk = lax.shift_right_logical(rv, 17) == g
            pc = jnp.sum(mk.astype(jnp.int32))

            @pl.when(pc > 0)
            def _store():
                pos = o + jnp.cumsum(mk.astype(jnp.int32)) - 1
                iv2 = bucket_i[pl.ds(qo, 16)]
                plsc.store_scatter(bucket2_r, [pos], rv, mask=mk)
                plsc.store_scatter(bucket2_i, [pos], iv2, mask=mk)

            return o + pc, gc + pc

        off, gcnt = lax.fori_loop(0, nchunks, p1b, (off, 0))
        plsc.store_scatter(bucket2_r, [off + iota],
                           jnp.full((16,), _SENT, jnp.int32))
        goffs.append(off - gcnt)
        gchunks.append(lax.shift_right_logical(gcnt + 15, 4))
        off = lax.shift_right_logical(goffs[-1] + gcnt + 16 + 15, 4) * 16

    nt = lax.select(w < _NFULL - (_NFULL // _NW) * _NW,
                    _NFULL // _NW + 1, _NFULL // _NW)

    def chunk_slice(t):
        win = w + _NW * t
        r0 = pl.multiple_of(win * _WIN, 128)
        return table_hbm.at[:, :, pl.ds(r0, _WIN)]

    def fire(t):
        slot = lax.rem(t, 2)
        pltpu.async_copy(chunk_slice(t), chunks.at[slot], csem.at[slot])

    fire(0)

    carry = (0, 0, 0)
    for g in range(8):
        def wbody(tt, carry, g=g):
            t = g * 16 + tt
            win = w + _NW * t

            @pl.when(t + 1 < nt)
            def _prefetch():
                fire(t + 1)

            pltpu.make_async_copy(
                chunk_slice(t), chunks.at[lax.rem(t, 2)],
                csem.at[lax.rem(t, 2)]
            ).wait()
            flush = _make_flush(chunks.at[lax.rem(t, 2)], staging, ring_l,
                                ring_i, out_hbm, osem, iota)
            return _scan_window(win, goffs[g], gchunks[g], carry,
                                bucket2_r, bucket2_i, ring_l, ring_i,
                                flush, iota)

        n_g = jnp.clip(nt - g * 16, 0, 16)
        carry = lax.fori_loop(0, n_g, wbody, carry)

    pltpu.sync_copy(
        table_hbm.at[:, :, pl.ds(_TAILW * _WIN, _V - _TAILW * _WIN)], tailc
    )
    flush_t = _make_flush(tailc, staging, ring_l, ring_i, out_hbm, osem,
                          iota)
    carry = _scan_window(_TAILW, goffs[7], gchunks[7], carry,
                         bucket2_r, bucket2_i, ring_l, ring_i, flush_t,
                         iota)

    wr, rd, bctr = carry
    dv = jnp.zeros((16,), jnp.int32)
    for k, val in enumerate([cnt, wr, rd, bctr, nt, nchunks]):
        dv = jnp.where(iota == k, val, dv)
    ring_l[pl.ds(0, 16)] = dv
    pltpu.sync_copy(ring_l.at[pl.ds(0, 16)],
                    diag_hbm.at[pl.ds(pl.multiple_of(w * 16, 16), 16)])
    pltpu.sync_copy(diag_hbm.at[pl.ds(pl.multiple_of(w * 16, 16), 16)],
                    ring_i.at[pl.ds(0, 16)])


def kernel(input, index):
    table3 = jnp.reshape(jnp.swapaxes(input, 0, 1), (8, 8, _V))
    index = index.astype(jnp.int32)
    mesh = plsc.VectorSubcoreMesh(core_axis_name="c", subcore_axis_name="s")
    run = functools.partial(
        pl.kernel,
        mesh=mesh,
        out_type=(jax.ShapeDtypeStruct((_B * _D,), jnp.float32),
                  jax.ShapeDtypeStruct((_NW * 16,), jnp.int32)),
        scratch_types=[
            pltpu.VMEM((_B,), jnp.int32),
            pltpu.VMEM((_CAP,), jnp.int32),
            pltpu.VMEM((_CAP,), jnp.int32),
            pltpu.VMEM((_B + 256,), jnp.int32),
            pltpu.VMEM((_B + 256,), jnp.int32),
            pltpu.VMEM((2, 8, 8, _WIN), jnp.float32),
            pltpu.VMEM((8, 8, _V - _TAILW * _WIN), jnp.float32),
            pltpu.VMEM((64,), jnp.int32),
            pltpu.VMEM((64,), jnp.int32),
            pltpu.VMEM((6 * 16 * _D,), jnp.float32),
            pltpu.SemaphoreType.DMA((2,)),
            pltpu.SemaphoreType.DMA((8,)),
        ],
        compiler_params=pltpu.CompilerParams(needs_layout_passes=False),
    )(_gather_body)
    out1, _ = run(table3, index)
    return jnp.reshape(out1, (_B, _D))
